# baseline (speedup 1.0000x reference)
import functools

import jax
import jax.numpy as jnp
from jax import lax
from jax.experimental import pallas as pl
from jax.experimental.pallas import tpu as pltpu
from jax.experimental.pallas import tpu_sc as plsc

N = 10000
E = 160000
NB = 16
AF = 128
BF = 144
H = 256
DEPTH = 4
MOLSZ = 50


def _mm_in_body(x_ref, wi_ref, wh_ref, o_ref, m_ref, n_ref):
    r = jnp.dot(x_ref[...], wi_ref[...], preferred_element_type=jnp.float32)
    msg = jnp.maximum(r, 0.0)
    o_ref[...] = r
    m_ref[...] = msg
    n_ref[...] = -jnp.dot(msg, wh_ref[...], preferred_element_type=jnp.float32)


def _mm_in(x, w_i, w_h, be=1000):
    m = x.shape[0]
    return pl.pallas_call(
        _mm_in_body,
        grid=(m // be,),
        in_specs=[
            pl.BlockSpec((be, x.shape[1]), lambda i: (i, 0)),
            pl.BlockSpec((x.shape[1], H), lambda i: (0, 0)),
            pl.BlockSpec((H, H), lambda i: (0, 0)),
        ],
        out_specs=[
            pl.BlockSpec((be, H), lambda i: (i, 0)),
            pl.BlockSpec((be, H), lambda i: (i, 0)),
            pl.BlockSpec((be, H), lambda i: (i, 0)),
        ],
        out_shape=[
            jax.ShapeDtypeStruct((m, H), jnp.float32),
            jax.ShapeDtypeStruct((m, H), jnp.float32),
            jax.ShapeDtypeStruct((m, H), jnp.float32),
        ],
    )(x, w_i, w_h)


def _mm_h_body(x_ref, w_ref, o_ref):
    o_ref[...] = -jnp.dot(x_ref[...], w_ref[...],
                          preferred_element_type=jnp.float32)


def _mm_h_neg(msg, w, be=1000):
    m = msg.shape[0]
    return pl.pallas_call(
        _mm_h_body,
        grid=(m // be,),
        in_specs=[
            pl.BlockSpec((be, H), lambda i: (i, 0)),
            pl.BlockSpec((H, H), lambda i: (0, 0)),
        ],
        out_specs=pl.BlockSpec((be, H), lambda i: (i, 0)),
        out_shape=jax.ShapeDtypeStruct((m, H), jnp.float32),
    )(msg, w)


def _mm_plain_body(x_ref, w_ref, o_ref):
    o_ref[...] = jnp.dot(x_ref[...], w_ref[...],
                         preferred_element_type=jnp.float32)


def _mm_plain(x, w, be=1000):
    m = x.shape[0]
    return pl.pallas_call(
        _mm_plain_body,
        grid=(m // be,),
        in_specs=[
            pl.BlockSpec((be, H), lambda i: (i, 0)),
            pl.BlockSpec((H, H), lambda i: (0, 0)),
        ],
        out_specs=pl.BlockSpec((be, H), lambda i: (i, 0)),
        out_shape=jax.ShapeDtypeStruct((m, H), jnp.float32),
    )(x, w)


def _mm_out_body(fa_ref, am_ref, woa_ref, wom_ref, bo_ref, pool_ref, o_ref):
    hid = jnp.dot(fa_ref[...], woa_ref[...], preferred_element_type=jnp.float32)
    hid += jnp.dot(am_ref[...], wom_ref[...], preferred_element_type=jnp.float32)
    hid = jnp.maximum(hid + bo_ref[...], 0.0)
    o_ref[...] = jnp.dot(pool_ref[...], hid, preferred_element_type=jnp.float32)


def _mm_out(f_atoms, a_msg, w_o, b_o, ba=2000):
    m = f_atoms.shape[0]
    nmol_b = ba // MOLSZ
    w_oa = w_o[:AF]
    w_om = w_o[AF:]
    pool = jnp.kron(jnp.eye(nmol_b, dtype=jnp.float32),
                    jnp.full((1, MOLSZ), 1.0 / MOLSZ, dtype=jnp.float32))
    return pl.pallas_call(
        _mm_out_body,
        grid=(m // ba,),
        in_specs=[
            pl.BlockSpec((ba, AF), lambda i: (i, 0)),
            pl.BlockSpec((ba, H), lambda i: (i, 0)),
            pl.BlockSpec((AF, H), lambda i: (0, 0)),
            pl.BlockSpec((H, H), lambda i: (0, 0)),
            pl.BlockSpec((1, H), lambda i: (0, 0)),
            pl.BlockSpec((nmol_b, ba), lambda i: (0, 0)),
        ],
        out_specs=pl.BlockSpec((nmol_b, H), lambda i: (i, 0)),
        out_shape=jax.ShapeDtypeStruct((m // MOLSZ, H), jnp.float32),
    )(f_atoms, a_msg, w_oa, w_om, b_o.reshape(1, H), pool)


_NC, _NS, _L = 2, 16, 16
_NW = _NC * _NS
_MESH = plsc.VectorSubcoreMesh(core_axis_name="c", subcore_axis_name="s",
                               num_cores=_NC, num_subcores=_NS)

_CA = 4
_G1_ROWS = _CA * NB
_G1_NS = 4
_G1_B0 = 112
_G1_B1 = 48
_NPAD = 16 * (_G1_B0 + _G1_B1) * _CA
_G1_IDXPAD = (16 * _G1_B0 + 15 * _G1_B1 + _G1_B0) * _G1_ROWS

_CE = 40
_EPW0 = 7040
_EPW1 = 2960
_CH0 = _EPW0 // _CE
_CH1 = _EPW1 // _CE
_G2_IDXPAD = 16 * _EPW0 + 15 * _EPW1 + _EPW0


_G1_HR = _G1_ROWS // 2


@functools.partial(
    pl.kernel,
    out_type=jax.ShapeDtypeStruct((_NPAD, H), jnp.float32),
    mesh=_MESH,
    scratch_types=[
        pltpu.VMEM((_G1_B0 * _G1_ROWS,), jnp.int32),
    ] + [pltpu.VMEM((_G1_ROWS, H), jnp.float32)] * _G1_NS
      + [pltpu.VMEM((_CA, H), jnp.float32)] * _G1_NS
      + [pltpu.SemaphoreType.DMA] * (2 * _G1_NS),
)
def _sc_gathersum(m_hbm, idx_hbm, out_hbm, idxv, *bufs):
    gb = bufs[:_G1_NS]
    ob = bufs[_G1_NS:2 * _G1_NS]
    sg = bufs[2 * _G1_NS:3 * _G1_NS]
    so = bufs[3 * _G1_NS:4 * _G1_NS]
    cid = lax.axis_index("c")
    sid = lax.axis_index("s")
    nblk = jnp.where(cid == 0, _G1_B0, _G1_B1)
    bstart = jnp.where(cid == 0, sid * _G1_B0, 16 * _G1_B0 + sid * _G1_B1)
    pltpu.sync_copy(idx_hbm.at[pl.ds(bstart * _G1_ROWS, _G1_B0 * _G1_ROWS)],
                    idxv)

    def g_desc(k, s):
        return pltpu.make_async_copy(
            m_hbm.at[idxv.at[pl.ds(k * _G1_ROWS, _G1_ROWS)]], gb[s], sg[s])

    def o_desc(k, s):
        return pltpu.make_async_copy(
            ob[s], out_hbm.at[pl.ds((bstart + k) * _CA, _CA)], so[s])

    for s in range(_G1_NS):
        g_desc(s, s).start()

    @pl.loop(0, nblk, step=_G1_NS)
    def grp(c):
        for b in range(_G1_NS):
            k = c + b
            g_desc(k, b).wait()

            @pl.when(k >= _G1_NS)
            def _():
                o_desc(k - _G1_NS, b).wait()

            def atom(a, carry):
                base = a * NB
                for h in range(H // _L):
                    hs = pl.ds(h * _L, _L)
                    acc = gb[b][base, hs]
                    for j in range(1, NB):
                        acc = acc + gb[b][base + j, hs]
                    ob[b][a, hs] = acc
                return carry

            lax.fori_loop(0, _CA, atom, 0)
            o_desc(k, b).start()

            @pl.when(k + _G1_NS < nblk)
            def _():
                g_desc(k + _G1_NS, b).start()

    for s in range(_G1_NS):
        o_desc(nblk - _G1_NS + s, s).wait()


@functools.partial(
    pl.kernel,
    out_type=jax.ShapeDtypeStruct((E, H), jnp.float32),
    mesh=_MESH,
    scratch_types=[
        pltpu.VMEM((_EPW0,), jnp.int32),
        pltpu.VMEM((_EPW0,), jnp.int32),
        pltpu.VMEM((_CE, H), jnp.float32),
        pltpu.VMEM((_CE, H), jnp.float32),
        pltpu.VMEM((_CE, H), jnp.float32),
        pltpu.VMEM((_CE, H), jnp.float32),
        pltpu.VMEM((_CE, H), jnp.float32),
        pltpu.VMEM((_CE, H), jnp.float32),
        pltpu.VMEM((_CE, H), jnp.float32),
        pltpu.VMEM((_CE, H), jnp.float32),
        pltpu.SemaphoreType.DMA,
        pltpu.SemaphoreType.DMA,
        pltpu.SemaphoreType.DMA,
        pltpu.SemaphoreType.DMA,
        pltpu.SemaphoreType.DMA,
        pltpu.SemaphoreType.DMA,
        pltpu.SemaphoreType.DMA,
        pltpu.SemaphoreType.DMA,
    ],
)
def _sc_combine(inp0_hbm, ah_hbm, mhn_hbm, b2a_hbm, b2revb_hbm, out_hbm,
                av, rv, ibuf0, ibuf1, abuf0, abuf1, mbuf0, mbuf1,
                obuf0, obuf1, si0, si1, sa0, sa1, sm0, sm1, so0, so1):
    cid = lax.axis_index("c")
    sid = lax.axis_index("s")
    nch = jnp.where(cid == 0, _CH0, _CH1)
    ebase = jnp.where(cid == 0, sid * _EPW0, 16 * _EPW0 + sid * _EPW1)
    pltpu.sync_copy(b2a_hbm.at[pl.ds(ebase, _EPW0)], av)
    pltpu.sync_copy(b2revb_hbm.at[pl.ds(ebase, _EPW0)], rv)
    ib, ab, mb, obf = (ibuf0, ibuf1), (abuf0, abuf1), (mbuf0, mbuf1), (obuf0, obuf1)
    si, sa, sm, so = (si0, si1), (sa0, sa1), (sm0, sm1), (so0, so1)

    def in_descs(k, s):
        eb = ebase + k * _CE
        return (
            pltpu.make_async_copy(inp0_hbm.at[pl.ds(eb, _CE)], ib[s], si[s]),
            pltpu.make_async_copy(ah_hbm.at[av.at[pl.ds(k * _CE, _CE)]],
                                  ab[s], sa[s]),
            pltpu.make_async_copy(mhn_hbm.at[rv.at[pl.ds(k * _CE, _CE)]],
                                  mb[s], sm[s]),
        )

    def o_desc(k, s):
        return pltpu.make_async_copy(
            obf[s], out_hbm.at[pl.ds(ebase + k * _CE, _CE)], so[s])

    def body(k, s):
        for d in in_descs(k, s):
            d.wait()

        @pl.when(k >= 2)
        def _():
            o_desc(k - 2, s).wait()

        def eloop(e, carry):
            for h in range(H // _L):
                hs = pl.ds(h * _L, _L)
                obf[s][e, hs] = jnp.maximum(
                    ib[s][e, hs] + ab[s][e, hs] + mb[s][e, hs], 0.0)
            return carry

        lax.fori_loop(0, _CE, eloop, 0)
        o_desc(k, s).start()

        @pl.when(k + 2 < nch)
        def _():
            for d in in_descs(k + 2, s):
                d.start()

    for d in in_descs(0, 0):
        d.start()
    for d in in_descs(1, 1):
        d.start()

    @pl.loop(0, nch, step=2)
    def pair(c):
        body(c, 0)
        body(c + 1, 1)

    o_desc(nch - 2, 0).wait()
    o_desc(nch - 1, 1).wait()


def _gathersum(msg, a2b_flat):
    idx_pad = jnp.pad(a2b_flat, (0, _G1_IDXPAD - a2b_flat.shape[0]))
    return _sc_gathersum(msg, idx_pad)[:N]


def kernel(f_atoms_sv, f_bonds_sv, a2b_sv, b2a_sv, b2revb_sv,
           f_atoms_su, f_bonds_su, a2b_su, b2a_su, b2revb_su,
           W_i, W_h, W_o, b_o):
    enc = []
    for (fa, fb, a2b, b2a, b2revb) in (
        (f_atoms_sv, f_bonds_sv, a2b_sv, b2a_sv, b2revb_sv),
        (f_atoms_su, f_bonds_su, a2b_su, b2a_su, b2revb_su),
    ):
        a2b_flat = a2b.astype(jnp.int32).reshape(-1)
        b2a_p = jnp.pad(b2a, (0, _G2_IDXPAD - E))
        b2revb_p = jnp.pad(b2revb, (0, _G2_IDXPAD - E))
        inp0, msg, m_hn = _mm_in(fb, W_i, W_h)
        enc.append({"fa": fa, "a2b": a2b_flat, "b2a": b2a_p, "b2revb": b2revb_p,
                    "inp0": inp0, "msg": msg, "m_hn": m_hn})

    for it in range(DEPTH - 1):
        for s in enc:
            s["a_msg"] = _gathersum(s["msg"], s["a2b"])
        for s in enc:
            s["a_h"] = _mm_plain(s["a_msg"], W_h)
        for s in enc:
            s["msg"] = _sc_combine(s["inp0"], s["a_h"], s["m_hn"],
                                   s["b2a"], s["b2revb"])
        if it < DEPTH - 2:
            for s in enc:
                s["m_hn"] = _mm_h_neg(s["msg"], W_h)

    for s in enc:
        s["a_msg"] = _gathersum(s["msg"], s["a2b"])
    mols = [_mm_out(s["fa"], s["a_msg"], W_o, b_o) for s in enc]
    return jnp.concatenate(mols, axis=1)


# speedup vs baseline: 1.0890x; 1.0890x over previous
import functools

import jax
import jax.numpy as jnp
from jax import lax
from jax.experimental import pallas as pl
from jax.experimental.pallas import tpu as pltpu
from jax.experimental.pallas import tpu_sc as plsc

N = 10000
E = 160000
NB = 16
AF = 128
BF = 144
H = 256
DEPTH = 4
MOLSZ = 50


def _mm_in_body(x_ref, wi_ref, wh_ref, o_ref, m_ref, n_ref):
    r = jnp.dot(x_ref[...], wi_ref[...], preferred_element_type=jnp.float32)
    msg = jnp.maximum(r, 0.0)
    o_ref[...] = r
    m_ref[...] = msg
    n_ref[...] = -jnp.dot(msg, wh_ref[...], preferred_element_type=jnp.float32)


def _mm_in(x, w_i, w_h, be=1000):
    m = x.shape[0]
    return pl.pallas_call(
        _mm_in_body,
        grid=(m // be,),
        in_specs=[
            pl.BlockSpec((be, x.shape[1]), lambda i: (i, 0)),
            pl.BlockSpec((x.shape[1], H), lambda i: (0, 0)),
            pl.BlockSpec((H, H), lambda i: (0, 0)),
        ],
        out_specs=[
            pl.BlockSpec((be, H), lambda i: (i, 0)),
            pl.BlockSpec((be, H), lambda i: (i, 0)),
            pl.BlockSpec((be, H), lambda i: (i, 0)),
        ],
        out_shape=[
            jax.ShapeDtypeStruct((m, H), jnp.float32),
            jax.ShapeDtypeStruct((m, H), jnp.float32),
            jax.ShapeDtypeStruct((m, H), jnp.float32),
        ],
    )(x, w_i, w_h)


def _mm_h_body(x_ref, w_ref, o_ref):
    o_ref[...] = -jnp.dot(x_ref[...], w_ref[...],
                          preferred_element_type=jnp.float32)


def _mm_h_neg(msg, w, be=1000):
    m = msg.shape[0]
    return pl.pallas_call(
        _mm_h_body,
        grid=(m // be,),
        in_specs=[
            pl.BlockSpec((be, H), lambda i: (i, 0)),
            pl.BlockSpec((H, H), lambda i: (0, 0)),
        ],
        out_specs=pl.BlockSpec((be, H), lambda i: (i, 0)),
        out_shape=jax.ShapeDtypeStruct((m, H), jnp.float32),
    )(msg, w)


def _mm_plain_body(x_ref, w_ref, o_ref):
    o_ref[...] = jnp.dot(x_ref[...], w_ref[...],
                         preferred_element_type=jnp.float32)


def _mm_plain(x, w, be=1000):
    m = x.shape[0]
    return pl.pallas_call(
        _mm_plain_body,
        grid=(m // be,),
        in_specs=[
            pl.BlockSpec((be, H), lambda i: (i, 0)),
            pl.BlockSpec((H, H), lambda i: (0, 0)),
        ],
        out_specs=pl.BlockSpec((be, H), lambda i: (i, 0)),
        out_shape=jax.ShapeDtypeStruct((m, H), jnp.float32),
    )(x, w)


def _mm_out_body(fa_ref, am_ref, woa_ref, wom_ref, bo_ref, pool_ref, o_ref):
    hid = jnp.dot(fa_ref[...], woa_ref[...], preferred_element_type=jnp.float32)
    hid += jnp.dot(am_ref[...], wom_ref[...], preferred_element_type=jnp.float32)
    hid = jnp.maximum(hid + bo_ref[...], 0.0)
    o_ref[...] = jnp.dot(pool_ref[...], hid, preferred_element_type=jnp.float32)


def _mm_out(f_atoms, a_msg, w_o, b_o, ba=2000):
    m = f_atoms.shape[0]
    nmol_b = ba // MOLSZ
    w_oa = w_o[:AF]
    w_om = w_o[AF:]
    pool = jnp.kron(jnp.eye(nmol_b, dtype=jnp.float32),
                    jnp.full((1, MOLSZ), 1.0 / MOLSZ, dtype=jnp.float32))
    return pl.pallas_call(
        _mm_out_body,
        grid=(m // ba,),
        in_specs=[
            pl.BlockSpec((ba, AF), lambda i: (i, 0)),
            pl.BlockSpec((ba, H), lambda i: (i, 0)),
            pl.BlockSpec((AF, H), lambda i: (0, 0)),
            pl.BlockSpec((H, H), lambda i: (0, 0)),
            pl.BlockSpec((1, H), lambda i: (0, 0)),
            pl.BlockSpec((nmol_b, ba), lambda i: (0, 0)),
        ],
        out_specs=pl.BlockSpec((nmol_b, H), lambda i: (i, 0)),
        out_shape=jax.ShapeDtypeStruct((m // MOLSZ, H), jnp.float32),
    )(f_atoms, a_msg, w_oa, w_om, b_o.reshape(1, H), pool)


_NC, _NS, _L = 2, 16, 16
_NW = _NC * _NS
_MESH = plsc.VectorSubcoreMesh(core_axis_name="c", subcore_axis_name="s",
                               num_cores=_NC, num_subcores=_NS)

_CA = 4
_G1_ROWS = _CA * NB
_G1_NS = 4
_G1_B0 = 112
_G1_B1 = 48
_NPAD = 16 * (_G1_B0 + _G1_B1) * _CA
_G1_IDXPAD = (16 * _G1_B0 + 15 * _G1_B1 + _G1_B0) * _G1_ROWS

_EPW = E // _NW
_CE = 40
_G2_CH = _EPW // _CE


_G1_HR = _G1_ROWS // 2


@functools.partial(
    pl.kernel,
    out_type=jax.ShapeDtypeStruct((_NPAD, H), jnp.float32),
    mesh=_MESH,
    scratch_types=[
        pltpu.VMEM((_G1_B0 * _G1_ROWS,), jnp.int32),
    ] + [pltpu.VMEM((_G1_ROWS, H), jnp.float32)] * _G1_NS
      + [pltpu.VMEM((_CA, H), jnp.float32)] * _G1_NS
      + [pltpu.SemaphoreType.DMA] * (2 * _G1_NS),
)
def _sc_gathersum(m_hbm, idx_hbm, out_hbm, idxv, *bufs):
    gb = bufs[:_G1_NS]
    ob = bufs[_G1_NS:2 * _G1_NS]
    sg = bufs[2 * _G1_NS:3 * _G1_NS]
    so = bufs[3 * _G1_NS:4 * _G1_NS]
    cid = lax.axis_index("c")
    sid = lax.axis_index("s")
    nblk = jnp.where(cid == 0, _G1_B0, _G1_B1)
    bstart = jnp.where(cid == 0, sid * _G1_B0, 16 * _G1_B0 + sid * _G1_B1)
    pltpu.sync_copy(idx_hbm.at[pl.ds(bstart * _G1_ROWS, _G1_B0 * _G1_ROWS)],
                    idxv)

    def g_desc(k, s):
        return pltpu.make_async_copy(
            m_hbm.at[idxv.at[pl.ds(k * _G1_ROWS, _G1_ROWS)]], gb[s], sg[s])

    def o_desc(k, s):
        return pltpu.make_async_copy(
            ob[s], out_hbm.at[pl.ds((bstart + k) * _CA, _CA)], so[s])

    for s in range(_G1_NS):
        g_desc(s, s).start()

    @pl.loop(0, nblk, step=_G1_NS)
    def grp(c):
        for b in range(_G1_NS):
            k = c + b
            g_desc(k, b).wait()

            @pl.when(k >= _G1_NS)
            def _():
                o_desc(k - _G1_NS, b).wait()

            def atom(a, carry):
                base = a * NB
                for h in range(H // _L):
                    hs = pl.ds(h * _L, _L)
                    acc = gb[b][base, hs]
                    for j in range(1, NB):
                        acc = acc + gb[b][base + j, hs]
                    ob[b][a, hs] = acc
                return carry

            lax.fori_loop(0, _CA, atom, 0)
            o_desc(k, b).start()

            @pl.when(k + _G1_NS < nblk)
            def _():
                g_desc(k + _G1_NS, b).start()

    for s in range(_G1_NS):
        o_desc(nblk - _G1_NS + s, s).wait()


@functools.partial(
    pl.kernel,
    out_type=jax.ShapeDtypeStruct((E, H), jnp.float32),
    mesh=_MESH,
    scratch_types=[
        pltpu.VMEM((_EPW,), jnp.int32),
        pltpu.VMEM((_EPW,), jnp.int32),
        pltpu.VMEM((_CE, H), jnp.float32),
        pltpu.VMEM((_CE, H), jnp.float32),
        pltpu.VMEM((_CE, H), jnp.float32),
        pltpu.VMEM((_CE, H), jnp.float32),
        pltpu.VMEM((_CE, H), jnp.float32),
        pltpu.VMEM((_CE, H), jnp.float32),
        pltpu.VMEM((_CE, H), jnp.float32),
        pltpu.VMEM((_CE, H), jnp.float32),
        pltpu.SemaphoreType.DMA,
        pltpu.SemaphoreType.DMA,
        pltpu.SemaphoreType.DMA,
        pltpu.SemaphoreType.DMA,
        pltpu.SemaphoreType.DMA,
        pltpu.SemaphoreType.DMA,
        pltpu.SemaphoreType.DMA,
        pltpu.SemaphoreType.DMA,
    ],
)
def _sc_combine(inp0_hbm, ah_hbm, mhn_hbm, b2a_hbm, b2revb_hbm, out_hbm,
                av, rv, ibuf0, ibuf1, abuf0, abuf1, mbuf0, mbuf1,
                obuf0, obuf1, si0, si1, sa0, sa1, sm0, sm1, so0, so1):
    wid = lax.axis_index("s") * _NC + lax.axis_index("c")
    ebase = wid * _EPW
    pltpu.sync_copy(b2a_hbm.at[pl.ds(ebase, _EPW)], av)
    pltpu.sync_copy(b2revb_hbm.at[pl.ds(ebase, _EPW)], rv)
    ib, ab, mb, obf = (ibuf0, ibuf1), (abuf0, abuf1), (mbuf0, mbuf1), (obuf0, obuf1)
    si, sa, sm, so = (si0, si1), (sa0, sa1), (sm0, sm1), (so0, so1)

    def in_descs(k, s):
        eb = ebase + k * _CE
        return (
            pltpu.make_async_copy(inp0_hbm.at[pl.ds(eb, _CE)], ib[s], si[s]),
            pltpu.make_async_copy(ah_hbm.at[av.at[pl.ds(k * _CE, _CE)]],
                                  ab[s], sa[s]),
            pltpu.make_async_copy(mhn_hbm.at[rv.at[pl.ds(k * _CE, _CE)]],
                                  mb[s], sm[s]),
        )

    def o_desc(k, s):
        return pltpu.make_async_copy(
            obf[s], out_hbm.at[pl.ds(ebase + k * _CE, _CE)], so[s])

    def body(k, s):
        for d in in_descs(k, s):
            d.wait()

        @pl.when(k >= 2)
        def _():
            o_desc(k - 2, s).wait()

        def eloop(e, carry):
            for h in range(H // _L):
                hs = pl.ds(h * _L, _L)
                obf[s][e, hs] = jnp.maximum(
                    ib[s][e, hs] + ab[s][e, hs] + mb[s][e, hs], 0.0)
            return carry

        lax.fori_loop(0, _CE, eloop, 0)
        o_desc(k, s).start()

        @pl.when(k + 2 < _G2_CH)
        def _():
            for d in in_descs(k + 2, s):
                d.start()

    for d in in_descs(0, 0):
        d.start()
    for d in in_descs(1, 1):
        d.start()

    body(0, 0)

    @pl.loop(1, _G2_CH, step=2)
    def pair(c):
        body(c, 1)
        body(c + 1, 0)

    o_desc(_G2_CH - 2, 1).wait()
    o_desc(_G2_CH - 1, 0).wait()


def _gathersum(msg, a2b_flat):
    idx_pad = jnp.pad(a2b_flat, (0, _G1_IDXPAD - a2b_flat.shape[0]))
    return _sc_gathersum(msg, idx_pad)[:N]


def kernel(f_atoms_sv, f_bonds_sv, a2b_sv, b2a_sv, b2revb_sv,
           f_atoms_su, f_bonds_su, a2b_su, b2a_su, b2revb_su,
           W_i, W_h, W_o, b_o):
    enc = []
    for (fa, fb, a2b, b2a, b2revb) in (
        (f_atoms_sv, f_bonds_sv, a2b_sv, b2a_sv, b2revb_sv),
        (f_atoms_su, f_bonds_su, a2b_su, b2a_su, b2revb_su),
    ):
        a2b_flat = a2b.astype(jnp.int32).reshape(-1)
        inp0, msg, m_hn = _mm_in(fb, W_i, W_h)
        enc.append({"fa": fa, "a2b": a2b_flat, "b2a": b2a, "b2revb": b2revb,
                    "inp0": inp0, "msg": msg, "m_hn": m_hn})

    for it in range(DEPTH - 1):
        for s in enc:
            s["a_msg"] = _gathersum(s["msg"], s["a2b"])
        for s in enc:
            s["a_h"] = _mm_plain(s["a_msg"], W_h)
        for s in enc:
            s["msg"] = _sc_combine(s["inp0"], s["a_h"], s["m_hn"],
                                   s["b2a"], s["b2revb"])
        if it < DEPTH - 2:
            for s in enc:
                s["m_hn"] = _mm_h_neg(s["msg"], W_h)

    for s in enc:
        s["a_msg"] = _gathersum(s["msg"], s["a2b"])
    mols = [_mm_out(s["fa"], s["a_msg"], W_o, b_o) for s in enc]
    return jnp.concatenate(mols, axis=1)
